# baseline (speedup 1.0000x reference)
import jax
import jax.numpy as jnp
from jax.experimental import pallas as pl
from jax.experimental.pallas import tpu as pltpu

_CHUNK = 2048
_N_BUF = 4


def _pipeline_kernel(x_hbm, w_ref, b_ref, o_hbm, xbufs, obufs, in_sems, out_sems):
    tokens = x_hbm.shape[0]
    n_chunks = tokens // _CHUNK

    def in_copy(i):
        slot = i % _N_BUF
        return pltpu.make_async_copy(
            x_hbm.at[pl.ds(i * _CHUNK, _CHUNK), :],
            xbufs.at[slot],
            in_sems.at[slot],
        )

    def out_copy(i):
        slot = i % _N_BUF
        return pltpu.make_async_copy(
            obufs.at[slot],
            o_hbm.at[pl.ds(i * _CHUNK, _CHUNK), :],
            out_sems.at[slot],
        )

    for i in range(min(_N_BUF, n_chunks)):
        in_copy(i).start()

    w = w_ref[...]
    b = b_ref[...]
    for i in range(n_chunks):
        slot = i % _N_BUF
        in_copy(i).wait()
        acc = jnp.dot(xbufs[slot], w, preferred_element_type=jnp.float32)
        if i >= _N_BUF:
            out_copy(i - _N_BUF).wait()
        obufs[slot] = jax.nn.sigmoid(acc + b)
        out_copy(i).start()
        nxt = i + _N_BUF
        if nxt < n_chunks:
            in_copy(nxt).start()
    for i in range(max(0, n_chunks - _N_BUF), n_chunks):
        out_copy(i).wait()


def kernel(x, W_heads, b_heads, W_free, b_free):
    tokens, d = x.shape
    n_comp, _, comp_size = W_heads.shape
    n_out = n_comp * comp_size + W_free.shape[1]
    W_all = jnp.concatenate(
        [jnp.transpose(W_heads, (1, 0, 2)).reshape(d, n_comp * comp_size), W_free],
        axis=1,
    )
    b_all = jnp.concatenate([b_heads.reshape(-1), b_free])[None, :]

    return pl.pallas_call(
        _pipeline_kernel,
        in_specs=[
            pl.BlockSpec(memory_space=pltpu.MemorySpace.HBM),
            pl.BlockSpec(memory_space=pltpu.MemorySpace.VMEM),
            pl.BlockSpec(memory_space=pltpu.MemorySpace.VMEM),
        ],
        out_specs=pl.BlockSpec(memory_space=pltpu.MemorySpace.HBM),
        out_shape=jax.ShapeDtypeStruct((tokens, n_out), x.dtype),
        scratch_shapes=[
            pltpu.VMEM((_N_BUF, _CHUNK, d), jnp.float32),
            pltpu.VMEM((_N_BUF, _CHUNK, n_out), jnp.float32),
            pltpu.SemaphoreType.DMA((_N_BUF,)),
            pltpu.SemaphoreType.DMA((_N_BUF,)),
        ],
    )(x, W_all, b_all)


# speedup vs baseline: 1.0690x; 1.0690x over previous
import jax
import jax.numpy as jnp
from jax.experimental import pallas as pl
from jax.experimental.pallas import tpu as pltpu

_BLOCK_M = 2048


def _fused_head_kernel(x_ref, w_ref, b_ref, o_ref):
    acc = jnp.dot(x_ref[...], w_ref[...], preferred_element_type=jnp.float32)
    o_ref[...] = jax.nn.sigmoid(acc + b_ref[...])


def kernel(x, W_heads, b_heads, W_free, b_free):
    tokens, d = x.shape
    n_comp, _, comp_size = W_heads.shape
    n_out = n_comp * comp_size + W_free.shape[1]
    W_all = jnp.concatenate(
        [jnp.transpose(W_heads, (1, 0, 2)).reshape(d, n_comp * comp_size), W_free],
        axis=1,
    )
    b_all = jnp.concatenate([b_heads.reshape(-1), b_free])[None, :]

    bm = min(_BLOCK_M, tokens)
    return pl.pallas_call(
        _fused_head_kernel,
        grid=(pl.cdiv(tokens, bm),),
        in_specs=[
            pl.BlockSpec((bm, d), lambda i: (i, 0)),
            pl.BlockSpec((d, n_out), lambda i: (0, 0)),
            pl.BlockSpec((1, n_out), lambda i: (0, 0)),
        ],
        out_specs=pl.BlockSpec((bm, n_out), lambda i: (i, 0)),
        out_shape=jax.ShapeDtypeStruct((tokens, n_out), x.dtype),
        compiler_params=pltpu.CompilerParams(
            dimension_semantics=("parallel",),
        ),
    )(x, W_all, b_all)
